# baseline (speedup 1.0000x reference)
import jax
import jax.numpy as jnp
from jax.experimental import pallas as pl
from jax.experimental.pallas import tpu as pltpu

_K = 2048
_ROWS_PER_BLOCK = 8


def _ksparse_block(x_ref, o_ref):
    x = x_ref[...]
    bits = jax.lax.bitcast_convert_type(x, jnp.int32)
    key = jnp.where(bits >= 0, bits, bits ^ jnp.int32(0x7FFFFFFF))
    rows = x.shape[0]
    prefix = jnp.full((rows, 1), jnp.int32(-(2**31)), jnp.int32)
    for bit in range(31, -1, -1):
        step = jnp.int32(-(2**31)) if bit == 31 else jnp.int32(1 << bit)
        cand = prefix + step
        cnt = jnp.sum((key >= cand).astype(jnp.int32), axis=-1, keepdims=True)
        prefix = jnp.where(cnt >= _K, cand, prefix)
    thr_bits = jnp.where(prefix >= 0, prefix, prefix ^ jnp.int32(0x7FFFFFFF))
    thr = jax.lax.bitcast_convert_type(thr_bits, jnp.float32)
    o_ref[...] = jnp.where(x >= thr, x, jnp.float32(0.0))


def kernel(inputs, k):
    del k
    n_rows, n_cols = inputs.shape
    r = _ROWS_PER_BLOCK
    return pl.pallas_call(
        _ksparse_block,
        grid=(n_rows // r,),
        in_specs=[pl.BlockSpec((r, n_cols), lambda i: (i, 0))],
        out_specs=pl.BlockSpec((r, n_cols), lambda i: (i, 0)),
        out_shape=jax.ShapeDtypeStruct(inputs.shape, inputs.dtype),
        compiler_params=pltpu.CompilerParams(
            dimension_semantics=("parallel",),
        ),
    )(inputs)


# speedup vs baseline: 1.8923x; 1.8923x over previous
import jax
import jax.numpy as jnp
from jax.experimental import pallas as pl
from jax.experimental.pallas import tpu as pltpu

_K = 2048
_ROWS_PER_BLOCK = 16


def _ksparse_block(x_ref, o_ref):
    x = x_ref[...]
    bits = jax.lax.bitcast_convert_type(x, jnp.int32)
    key = jnp.where(bits >= 0, bits, bits ^ jnp.int32(0x7FFFFFFF))
    rows = x.shape[0]
    prefix = jnp.full((rows, 1), jnp.int32(-(2**31)), jnp.int32)
    for bit in range(31, -1, -1):
        step = jnp.int32(-(2**31)) if bit == 31 else jnp.int32(1 << bit)
        cand = prefix + step
        ind = (key >= cand).astype(jnp.int32)
        w = ind.shape[-1]
        while w > 128:
            half = w // 2
            ind = ind[:, :half] + ind[:, half:]
            w = half
        cnt = jnp.sum(ind, axis=-1, keepdims=True)
        prefix = jnp.where(cnt >= _K, cand, prefix)
    thr_bits = jnp.where(prefix >= 0, prefix, prefix ^ jnp.int32(0x7FFFFFFF))
    thr = jax.lax.bitcast_convert_type(thr_bits, jnp.float32)
    o_ref[...] = jnp.where(x >= thr, x, jnp.float32(0.0))


def kernel(inputs, k):
    del k
    n_rows, n_cols = inputs.shape
    r = _ROWS_PER_BLOCK
    return pl.pallas_call(
        _ksparse_block,
        grid=(n_rows // r,),
        in_specs=[pl.BlockSpec((r, n_cols), lambda i: (i, 0))],
        out_specs=pl.BlockSpec((r, n_cols), lambda i: (i, 0)),
        out_shape=jax.ShapeDtypeStruct(inputs.shape, inputs.dtype),
        compiler_params=pltpu.CompilerParams(
            dimension_semantics=("parallel",),
        ),
    )(inputs)


# speedup vs baseline: 2.1986x; 1.1619x over previous
import jax
import jax.numpy as jnp
from jax.experimental import pallas as pl
from jax.experimental.pallas import tpu as pltpu

_K = 2048
_ROWS_PER_BLOCK = 32


def _ksparse_block(x_ref, o_ref):
    x = x_ref[...]
    bits = jax.lax.bitcast_convert_type(x, jnp.int32)
    key = jnp.where(bits >= 0, bits, bits ^ jnp.int32(0x7FFFFFFF))
    rows = x.shape[0]
    prefix = jnp.full((rows, 1), jnp.int32(-(2**31)), jnp.int32)
    for bit in range(31, -1, -1):
        step = jnp.int32(-(2**31)) if bit == 31 else jnp.int32(1 << bit)
        cand = prefix + step
        ind = (key >= cand).astype(jnp.int32)
        w = ind.shape[-1]
        while w > 128:
            half = w // 2
            ind = ind[:, :half] + ind[:, half:]
            w = half
        cnt = jnp.sum(ind, axis=-1, keepdims=True)
        prefix = jnp.where(cnt >= _K, cand, prefix)
    thr_bits = jnp.where(prefix >= 0, prefix, prefix ^ jnp.int32(0x7FFFFFFF))
    thr = jax.lax.bitcast_convert_type(thr_bits, jnp.float32)
    o_ref[...] = jnp.where(x >= thr, x, jnp.float32(0.0))


def kernel(inputs, k):
    del k
    n_rows, n_cols = inputs.shape
    r = _ROWS_PER_BLOCK
    return pl.pallas_call(
        _ksparse_block,
        grid=(n_rows // r,),
        in_specs=[pl.BlockSpec((r, n_cols), lambda i: (i, 0))],
        out_specs=pl.BlockSpec((r, n_cols), lambda i: (i, 0)),
        out_shape=jax.ShapeDtypeStruct(inputs.shape, inputs.dtype),
        compiler_params=pltpu.CompilerParams(
            dimension_semantics=("parallel",),
        ),
    )(inputs)


# speedup vs baseline: 2.4073x; 1.0949x over previous
import jax
import jax.numpy as jnp
from jax.experimental import pallas as pl
from jax.experimental.pallas import tpu as pltpu

_K = 2048
_ROWS_PER_BLOCK = 64


def _ksparse_block(x_ref, o_ref):
    x = x_ref[...]
    bits = jax.lax.bitcast_convert_type(x, jnp.int32)
    key = jnp.where(bits >= 0, bits, bits ^ jnp.int32(0x7FFFFFFF))
    rows = x.shape[0]
    prefix = jnp.full((rows, 1), jnp.int32(-(2**31)), jnp.int32)
    for bit in range(31, -1, -1):
        step = jnp.int32(-(2**31)) if bit == 31 else jnp.int32(1 << bit)
        cand = prefix + step
        ind = (key >= cand).astype(jnp.int32)
        w = ind.shape[-1]
        while w > 128:
            half = w // 2
            ind = ind[:, :half] + ind[:, half:]
            w = half
        cnt = jnp.sum(ind, axis=-1, keepdims=True)
        prefix = jnp.where(cnt >= _K, cand, prefix)
    thr_bits = jnp.where(prefix >= 0, prefix, prefix ^ jnp.int32(0x7FFFFFFF))
    thr = jax.lax.bitcast_convert_type(thr_bits, jnp.float32)
    o_ref[...] = jnp.where(x >= thr, x, jnp.float32(0.0))


def kernel(inputs, k):
    del k
    n_rows, n_cols = inputs.shape
    r = _ROWS_PER_BLOCK
    return pl.pallas_call(
        _ksparse_block,
        grid=(n_rows // r,),
        in_specs=[pl.BlockSpec((r, n_cols), lambda i: (i, 0))],
        out_specs=pl.BlockSpec((r, n_cols), lambda i: (i, 0)),
        out_shape=jax.ShapeDtypeStruct(inputs.shape, inputs.dtype),
        compiler_params=pltpu.CompilerParams(
            dimension_semantics=("parallel",),
        ),
    )(inputs)
